# baseline (speedup 1.0000x reference)
import functools

import jax
import jax.numpy as jnp
from jax import lax
from jax.experimental import pallas as pl
from jax.experimental.pallas import tpu as pltpu
from jax.experimental.pallas import tpu_sc as plsc

_B = 8
_N = 2048
_PAY = 128
_TROWS = 2056
_NC, _NS = 2, 16
_NW = _NC * _NS
_RPW = (_B * _N) // _NW


def _front_kernel(obs_ref, obs_t_ref, w_ref, w_t_ref, a_src_ref, a_dst_ref,
                  pay_ref, d_ref, t_ref, sm_ref):
    obs = obs_ref[0]
    obs_t = obs_t_ref[0]
    h = jnp.dot(obs, w_ref[...], preferred_element_type=jnp.float32)

    vs_row = jnp.dot(a_src_ref[...], w_t_ref[...],
                     preferred_element_type=jnp.float32)
    vd_row = jnp.dot(a_dst_ref[...], w_t_ref[...],
                     preferred_element_type=jnp.float32)
    s_row = jnp.dot(vs_row, obs_t, preferred_element_type=jnp.float32)
    d_row = jnp.dot(vd_row, obs_t, preferred_element_type=jnp.float32)

    d_col = jnp.dot(h, jnp.transpose(a_dst_ref[...]),
                    preferred_element_type=jnp.float32)

    m = jnp.max(d_row)
    p_col = jnp.exp(d_col - m)
    q_col = jnp.exp(0.2 * (d_col - m))

    pay = jnp.concatenate(
        [p_col * h, p_col, q_col * h, q_col,
         jnp.zeros((_N, _PAY - 50), dtype=jnp.float32)], axis=1)
    pay_ref[...] = pay
    d_ref[0] = d_row
    t_ref[0] = -s_row
    sm_ref[0] = s_row + m


def _sort_kernel(d_ref, sd_ref, perm_ref):
    v = d_ref[...]
    li = lax.broadcasted_iota(jnp.int32, (_B, _N), 1)
    idx = li
    for size_log in range(1, 12):
        size = 1 << size_log
        for dist_log in range(size_log - 1, -1, -1):
            dist = 1 << dist_log
            lower = (li & dist) == 0
            asc = (li & size) == 0
            want_small = lower == asc
            pv = jnp.where(lower, jnp.roll(v, -dist, axis=1),
                           jnp.roll(v, dist, axis=1))
            pi = jnp.where(lower, jnp.roll(idx, -dist, axis=1),
                           jnp.roll(idx, dist, axis=1))
            take_p = pv < v
            small_v = jnp.where(take_p, pv, v)
            large_v = jnp.where(take_p, v, pv)
            small_i = jnp.where(take_p, pi, idx)
            large_i = jnp.where(take_p, idx, pi)
            v = jnp.where(want_small, small_v, large_v)
            idx = jnp.where(want_small, small_i, large_i)
    sd_ref[...] = v
    perm_ref[...] = idx + lax.broadcasted_iota(jnp.int32, (_B, _N), 0) * _N


def _gather_rows_body(pay_hbm, perm_hbm, out_hbm, idx_v, rows_v, sem):
    wid = lax.axis_index("s") * _NC + lax.axis_index("c")
    base = wid * _RPW
    for c in range(_RPW // 128):
        off = base + c * 128
        pltpu.sync_copy(perm_hbm.at[pl.ds(off, 128)], idx_v)
        pltpu.async_copy(pay_hbm.at[idx_v], rows_v, sem).wait()
        pltpu.sync_copy(rows_v, out_hbm.at[pl.ds(off, 128)])


@functools.cache
def _gather_rows_sc():
    mesh = plsc.VectorSubcoreMesh(core_axis_name="c", subcore_axis_name="s")
    return pl.kernel(
        _gather_rows_body, mesh=mesh,
        out_type=jax.ShapeDtypeStruct((_B * _N, _PAY), jnp.float32),
        scratch_types=[
            pltpu.VMEM((128,), jnp.int32),
            pltpu.VMEM((128, _PAY), jnp.float32),
            pltpu.SemaphoreType.DMA,
        ],
        compiler_params=pltpu.CompilerParams(needs_layout_passes=False),
    )


def _cumsum_kernel(spay_ref, table_ref):
    c = spay_ref[0]
    sh = 1
    while sh < _N:
        c = c + jnp.concatenate(
            [jnp.zeros((sh, _PAY), dtype=jnp.float32), c[:-sh, :]], axis=0)
        sh *= 2
    table_ref[0] = jnp.concatenate(
        [jnp.zeros((1, _PAY), dtype=jnp.float32), c,
         jnp.zeros((_TROWS - _N - 1, _PAY), dtype=jnp.float32)], axis=0)


_SEARCH_STEPS = (1024, 512, 256, 128, 64, 32, 16, 8, 4, 2, 1)


def _search_gather_body(sd_hbm, t_hbm, table_hbm, out_hbm,
                        sd_v, t_v, rows_v, sem):
    wid = lax.axis_index("s") * _NC + lax.axis_index("c")
    b = wid // (_NW // _B)
    chunk = wid % (_NW // _B)
    qbase = b * _N + chunk * _RPW
    pltpu.sync_copy(sd_hbm.at[b], sd_v)
    pltpu.sync_copy(t_hbm.at[pl.ds(qbase, _RPW)], t_v)
    tbase = b * _TROWS
    for g in range(_RPW // 16):
        tq = t_v[pl.ds(g * 16, 16)]
        r = jnp.zeros((16,), jnp.int32)
        for step in _SEARCH_STEPS:
            probe = r + (step - 1)
            val = plsc.load_gather(sd_v, [probe])
            r = jnp.where(val < tq, r + step, r)
        pltpu.async_copy(table_hbm.at[r + tbase], rows_v, sem).wait()
        pltpu.sync_copy(rows_v, out_hbm.at[pl.ds(qbase + g * 16, 16)])


@functools.cache
def _search_gather_sc():
    mesh = plsc.VectorSubcoreMesh(core_axis_name="c", subcore_axis_name="s")
    return pl.kernel(
        _search_gather_body, mesh=mesh,
        out_type=jax.ShapeDtypeStruct((_B * _N, _PAY), jnp.float32),
        scratch_types=[
            pltpu.VMEM((_N,), jnp.float32),
            pltpu.VMEM((_RPW,), jnp.float32),
            pltpu.VMEM((16, _PAY), jnp.float32),
            pltpu.SemaphoreType.DMA,
        ],
        compiler_params=pltpu.CompilerParams(needs_layout_passes=False),
    )


def _combine_kernel(g_ref, tot_ref, sm_ref, w1_ref, b1_ref, w2_ref, b2_ref,
                    out_ref):
    g = g_ref[0]
    tot = tot_ref[0]
    sm = sm_ref[0]

    big_m = jnp.maximum(sm, 0.2 * sm)
    u = jnp.exp(sm - big_m)
    v = jnp.exp(0.2 * sm - big_m)

    a1 = tot[0:1, 0:25] - g[:, 0:25]
    a2 = g[:, 25:50]
    num = u * a1[:, 0:24] + v * a2[:, 0:24]
    den = u * a1[:, 24:25] + v * a2[:, 24:25]
    o = num / den
    e = jnp.where(o > 0, o, jnp.exp(o) - 1.0)
    mean = jnp.sum(e, axis=0, keepdims=True) * (1.0 / _N)
    z = jnp.maximum(jnp.dot(mean, w1_ref[...],
                            preferred_element_type=jnp.float32)
                    + b1_ref[...], 0.0)
    y = jnp.dot(z, w2_ref[...], preferred_element_type=jnp.float32) + b2_ref[...]
    out_ref[0] = jnp.broadcast_to(y, (8, 128))


def kernel(obs, W_gat, a_src, a_dst, W1, b1, W2, b2):
    obs_t = jnp.swapaxes(obs, 1, 2)
    a_src_row = a_src.reshape(1, 24)
    a_dst_row = a_dst.reshape(1, 24)
    b1_row = b1.reshape(1, 36)
    b2_s = b2.reshape(1, 1)
    d_in = obs.shape[2]

    pay, d8, t8, sm8 = pl.pallas_call(
        _front_kernel,
        grid=(_B,),
        in_specs=[
            pl.BlockSpec((1, _N, d_in), lambda b: (b, 0, 0)),
            pl.BlockSpec((1, d_in, _N), lambda b: (b, 0, 0)),
            pl.BlockSpec(W_gat.shape, lambda b: (0, 0)),
            pl.BlockSpec((24, d_in), lambda b: (0, 0)),
            pl.BlockSpec((1, 24), lambda b: (0, 0)),
            pl.BlockSpec((1, 24), lambda b: (0, 0)),
        ],
        out_specs=[
            pl.BlockSpec((_N, _PAY), lambda b: (b, 0)),
            pl.BlockSpec((1, 1, _N), lambda b: (b, 0, 0)),
            pl.BlockSpec((1, 1, _N), lambda b: (b, 0, 0)),
            pl.BlockSpec((1, 1, _N), lambda b: (b, 0, 0)),
        ],
        out_shape=[
            jax.ShapeDtypeStruct((_B * _N, _PAY), jnp.float32),
            jax.ShapeDtypeStruct((_B, 1, _N), jnp.float32),
            jax.ShapeDtypeStruct((_B, 1, _N), jnp.float32),
            jax.ShapeDtypeStruct((_B, 1, _N), jnp.float32),
        ],
        compiler_params=pltpu.CompilerParams(
            dimension_semantics=("arbitrary",)),
    )(obs, obs_t, W_gat, W_gat.T, a_src_row, a_dst_row)
    d8 = d8.reshape(_B, _N)
    t8 = t8.reshape(_B, _N)
    sm8 = sm8.reshape(_B, _N)

    sd8, perm = pl.pallas_call(
        _sort_kernel,
        out_shape=[
            jax.ShapeDtypeStruct((_B, _N), jnp.float32),
            jax.ShapeDtypeStruct((_B, _N), jnp.int32),
        ],
    )(d8)

    spay = _gather_rows_sc()(pay, perm.reshape(_B * _N))

    table = pl.pallas_call(
        _cumsum_kernel,
        grid=(_B,),
        in_specs=[pl.BlockSpec((1, _N, _PAY), lambda b: (b, 0, 0))],
        out_specs=pl.BlockSpec((1, _TROWS, _PAY), lambda b: (b, 0, 0)),
        out_shape=jax.ShapeDtypeStruct((_B, _TROWS, _PAY), jnp.float32),
        compiler_params=pltpu.CompilerParams(
            dimension_semantics=("arbitrary",)),
    )(spay.reshape(_B, _N, _PAY))

    g = _search_gather_sc()(sd8, t8.reshape(_B * _N),
                            table.reshape(_B * _TROWS, _PAY))

    padded = pl.pallas_call(
        _combine_kernel,
        grid=(_B,),
        in_specs=[
            pl.BlockSpec((1, _N, _PAY), lambda b: (b, 0, 0)),
            pl.BlockSpec((1, 8, _PAY), lambda b: (b, _N // 8, 0)),
            pl.BlockSpec((1, _N, 1), lambda b: (b, 0, 0)),
            pl.BlockSpec(W1.shape, lambda b: (0, 0)),
            pl.BlockSpec((1, 36), lambda b: (0, 0)),
            pl.BlockSpec(W2.shape, lambda b: (0, 0)),
            pl.BlockSpec((1, 1), lambda b: (0, 0)),
        ],
        out_specs=pl.BlockSpec((1, 8, 128), lambda b: (b, 0, 0)),
        out_shape=jax.ShapeDtypeStruct((_B, 8, 128), jnp.float32),
        compiler_params=pltpu.CompilerParams(
            dimension_semantics=("arbitrary",)),
    )(g.reshape(_B, _N, _PAY), table, sm8.reshape(_B, _N, 1),
      W1, b1_row, W2, b2_s)
    return padded[:, 0, :1]


# speedup vs baseline: 1.2025x; 1.2025x over previous
import functools

import jax
import jax.numpy as jnp
from jax import lax
from jax.experimental import pallas as pl
from jax.experimental.pallas import tpu as pltpu
from jax.experimental.pallas import tpu_sc as plsc

_B = 8
_N = 2048
_PAY = 128
_TROWS = 2056
_NC, _NS = 2, 16
_NW = _NC * _NS
_RPW = (_B * _N) // _NW


def _front_kernel(obs_ref, obs_t_ref, w_ref, w_t_ref, a_src_ref, a_dst_ref,
                  pay_ref, d_ref, t_ref, sm_ref):
    obs = obs_ref[0]
    obs_t = obs_t_ref[0]
    h = jnp.dot(obs, w_ref[...], preferred_element_type=jnp.float32)

    vs_row = jnp.dot(a_src_ref[...], w_t_ref[...],
                     preferred_element_type=jnp.float32)
    vd_row = jnp.dot(a_dst_ref[...], w_t_ref[...],
                     preferred_element_type=jnp.float32)
    s_row = jnp.dot(vs_row, obs_t, preferred_element_type=jnp.float32)
    d_row = jnp.dot(vd_row, obs_t, preferred_element_type=jnp.float32)

    d_col = jnp.dot(h, jnp.transpose(a_dst_ref[...]),
                    preferred_element_type=jnp.float32)

    m = jnp.max(d_row)
    p_col = jnp.exp(d_col - m)
    q_col = jnp.exp(0.2 * (d_col - m))

    pay = jnp.concatenate(
        [p_col * h, p_col, q_col * h, q_col,
         jnp.zeros((_N, _PAY - 50), dtype=jnp.float32)], axis=1)
    pay_ref[...] = pay
    d_ref[0] = d_row
    t_ref[0] = -s_row
    sm_ref[0] = s_row + m


def _sort_kernel(d_ref, sd_ref, perm_ref):
    v = d_ref[...]
    li = lax.broadcasted_iota(jnp.int32, (_B, _N), 1)
    idx = li
    for size_log in range(1, 12):
        size = 1 << size_log
        for dist_log in range(size_log - 1, -1, -1):
            dist = 1 << dist_log
            lower = (li & dist) == 0
            asc = (li & size) == 0
            want_small = lower == asc
            pv = jnp.where(lower, jnp.roll(v, -dist, axis=1),
                           jnp.roll(v, dist, axis=1))
            pi = jnp.where(lower, jnp.roll(idx, -dist, axis=1),
                           jnp.roll(idx, dist, axis=1))
            take_p = pv < v
            small_v = jnp.where(take_p, pv, v)
            large_v = jnp.where(take_p, v, pv)
            small_i = jnp.where(take_p, pi, idx)
            large_i = jnp.where(take_p, idx, pi)
            v = jnp.where(want_small, small_v, large_v)
            idx = jnp.where(want_small, small_i, large_i)
    sd_ref[...] = v
    perm_ref[...] = idx + lax.broadcasted_iota(jnp.int32, (_B, _N), 0) * _N


def _gather_rows_body(pay_hbm, perm_hbm, out_hbm, idx_v, rows_v, sem):
    wid = lax.axis_index("s") * _NC + lax.axis_index("c")
    base = wid * _RPW
    for c in range(_RPW // 128):
        off = base + c * 128
        pltpu.sync_copy(perm_hbm.at[pl.ds(off, 128)], idx_v)
        pltpu.async_copy(pay_hbm.at[idx_v], rows_v.at[pl.ds(c * 128, 128)],
                         sem).wait()
    pltpu.sync_copy(rows_v, out_hbm.at[pl.ds(base, _RPW)])


@functools.cache
def _gather_rows_sc():
    mesh = plsc.VectorSubcoreMesh(core_axis_name="c", subcore_axis_name="s")
    return pl.kernel(
        _gather_rows_body, mesh=mesh,
        out_type=jax.ShapeDtypeStruct((_B * _N, _PAY), jnp.float32),
        scratch_types=[
            pltpu.VMEM((128,), jnp.int32),
            pltpu.VMEM((_RPW, _PAY), jnp.float32),
            pltpu.SemaphoreType.DMA,
        ],
        compiler_params=pltpu.CompilerParams(needs_layout_passes=False),
    )


def _cumsum_kernel(spay_ref, table_ref):
    c = spay_ref[0]
    sh = 1
    while sh < _N:
        c = c + jnp.concatenate(
            [jnp.zeros((sh, _PAY), dtype=jnp.float32), c[:-sh, :]], axis=0)
        sh *= 2
    table_ref[0] = jnp.concatenate(
        [jnp.zeros((1, _PAY), dtype=jnp.float32), c,
         jnp.zeros((_TROWS - _N - 1, _PAY), dtype=jnp.float32)], axis=0)


_SEARCH_STEPS = (1024, 512, 256, 128, 64, 32, 16, 8, 4, 2, 1)


def _search_gather_body(sd_hbm, t_hbm, table_hbm, out_hbm,
                        sd_v, t_v, rows_v, sem):
    wid = lax.axis_index("s") * _NC + lax.axis_index("c")
    b = wid // (_NW // _B)
    chunk = wid % (_NW // _B)
    qbase = b * _N + chunk * _RPW
    pltpu.sync_copy(sd_hbm.at[b], sd_v)
    pltpu.sync_copy(t_hbm.at[pl.ds(qbase, _RPW)], t_v)
    tbase = b * _TROWS
    handles = []
    for g in range(_RPW // 16):
        tq = t_v[pl.ds(g * 16, 16)]
        r = jnp.zeros((16,), jnp.int32)
        for step in _SEARCH_STEPS:
            probe = r + (step - 1)
            val = plsc.load_gather(sd_v, [probe])
            r = jnp.where(val < tq, r + step, r)
        handles.append(pltpu.async_copy(table_hbm.at[r + tbase],
                                        rows_v.at[pl.ds(g * 16, 16)], sem))
        if len(handles) == 8:
            for hd in handles:
                hd.wait()
            handles = []
    for hd in handles:
        hd.wait()
    pltpu.sync_copy(rows_v, out_hbm.at[pl.ds(qbase, _RPW)])


@functools.cache
def _search_gather_sc():
    mesh = plsc.VectorSubcoreMesh(core_axis_name="c", subcore_axis_name="s")
    return pl.kernel(
        _search_gather_body, mesh=mesh,
        out_type=jax.ShapeDtypeStruct((_B * _N, _PAY), jnp.float32),
        scratch_types=[
            pltpu.VMEM((_N,), jnp.float32),
            pltpu.VMEM((_RPW,), jnp.float32),
            pltpu.VMEM((_RPW, _PAY), jnp.float32),
            pltpu.SemaphoreType.DMA,
        ],
        compiler_params=pltpu.CompilerParams(needs_layout_passes=False),
    )


def _combine_kernel(g_ref, tot_ref, sm_ref, w1_ref, b1_ref, w2_ref, b2_ref,
                    out_ref):
    g = g_ref[0]
    tot = tot_ref[0]
    sm = sm_ref[0]

    big_m = jnp.maximum(sm, 0.2 * sm)
    u = jnp.exp(sm - big_m)
    v = jnp.exp(0.2 * sm - big_m)

    a1 = tot[0:1, 0:25] - g[:, 0:25]
    a2 = g[:, 25:50]
    num = u * a1[:, 0:24] + v * a2[:, 0:24]
    den = u * a1[:, 24:25] + v * a2[:, 24:25]
    o = num / den
    e = jnp.where(o > 0, o, jnp.exp(o) - 1.0)
    mean = jnp.sum(e, axis=0, keepdims=True) * (1.0 / _N)
    z = jnp.maximum(jnp.dot(mean, w1_ref[...],
                            preferred_element_type=jnp.float32)
                    + b1_ref[...], 0.0)
    y = jnp.dot(z, w2_ref[...], preferred_element_type=jnp.float32) + b2_ref[...]
    out_ref[0] = jnp.broadcast_to(y, (8, 128))


def kernel(obs, W_gat, a_src, a_dst, W1, b1, W2, b2):
    obs_t = jnp.swapaxes(obs, 1, 2)
    a_src_row = a_src.reshape(1, 24)
    a_dst_row = a_dst.reshape(1, 24)
    b1_row = b1.reshape(1, 36)
    b2_s = b2.reshape(1, 1)
    d_in = obs.shape[2]

    pay, d8, t8, sm8 = pl.pallas_call(
        _front_kernel,
        grid=(_B,),
        in_specs=[
            pl.BlockSpec((1, _N, d_in), lambda b: (b, 0, 0)),
            pl.BlockSpec((1, d_in, _N), lambda b: (b, 0, 0)),
            pl.BlockSpec(W_gat.shape, lambda b: (0, 0)),
            pl.BlockSpec((24, d_in), lambda b: (0, 0)),
            pl.BlockSpec((1, 24), lambda b: (0, 0)),
            pl.BlockSpec((1, 24), lambda b: (0, 0)),
        ],
        out_specs=[
            pl.BlockSpec((_N, _PAY), lambda b: (b, 0)),
            pl.BlockSpec((1, 1, _N), lambda b: (b, 0, 0)),
            pl.BlockSpec((1, 1, _N), lambda b: (b, 0, 0)),
            pl.BlockSpec((1, 1, _N), lambda b: (b, 0, 0)),
        ],
        out_shape=[
            jax.ShapeDtypeStruct((_B * _N, _PAY), jnp.float32),
            jax.ShapeDtypeStruct((_B, 1, _N), jnp.float32),
            jax.ShapeDtypeStruct((_B, 1, _N), jnp.float32),
            jax.ShapeDtypeStruct((_B, 1, _N), jnp.float32),
        ],
        compiler_params=pltpu.CompilerParams(
            dimension_semantics=("arbitrary",)),
    )(obs, obs_t, W_gat, W_gat.T, a_src_row, a_dst_row)
    d8 = d8.reshape(_B, _N)
    t8 = t8.reshape(_B, _N)
    sm8 = sm8.reshape(_B, _N)

    sd8, perm = pl.pallas_call(
        _sort_kernel,
        out_shape=[
            jax.ShapeDtypeStruct((_B, _N), jnp.float32),
            jax.ShapeDtypeStruct((_B, _N), jnp.int32),
        ],
    )(d8)

    spay = _gather_rows_sc()(pay, perm.reshape(_B * _N))

    table = pl.pallas_call(
        _cumsum_kernel,
        grid=(_B,),
        in_specs=[pl.BlockSpec((1, _N, _PAY), lambda b: (b, 0, 0))],
        out_specs=pl.BlockSpec((1, _TROWS, _PAY), lambda b: (b, 0, 0)),
        out_shape=jax.ShapeDtypeStruct((_B, _TROWS, _PAY), jnp.float32),
        compiler_params=pltpu.CompilerParams(
            dimension_semantics=("arbitrary",)),
    )(spay.reshape(_B, _N, _PAY))

    g = _search_gather_sc()(sd8, t8.reshape(_B * _N),
                            table.reshape(_B * _TROWS, _PAY))

    padded = pl.pallas_call(
        _combine_kernel,
        grid=(_B,),
        in_specs=[
            pl.BlockSpec((1, _N, _PAY), lambda b: (b, 0, 0)),
            pl.BlockSpec((1, 8, _PAY), lambda b: (b, _N // 8, 0)),
            pl.BlockSpec((1, _N, 1), lambda b: (b, 0, 0)),
            pl.BlockSpec(W1.shape, lambda b: (0, 0)),
            pl.BlockSpec((1, 36), lambda b: (0, 0)),
            pl.BlockSpec(W2.shape, lambda b: (0, 0)),
            pl.BlockSpec((1, 1), lambda b: (0, 0)),
        ],
        out_specs=pl.BlockSpec((1, 8, 128), lambda b: (b, 0, 0)),
        out_shape=jax.ShapeDtypeStruct((_B, 8, 128), jnp.float32),
        compiler_params=pltpu.CompilerParams(
            dimension_semantics=("arbitrary",)),
    )(g.reshape(_B, _N, _PAY), table, sm8.reshape(_B, _N, 1),
      W1, b1_row, W2, b2_s)
    return padded[:, 0, :1]


# speedup vs baseline: 2.1387x; 1.7785x over previous
import jax
import jax.numpy as jnp
from jax.experimental import pallas as pl
from jax.experimental.pallas import tpu as pltpu

_N = 2048
_IB = 256


def _gat_value_kernel(obs_ref, obs_t_ref, w_gat_ref, w_gat_t_ref,
                      a_src_ref, a_dst_ref, w1_ref, b1_ref, w2_ref, b2_ref,
                      out_ref):
    obs = obs_ref[0]
    obs_t = obs_t_ref[0]

    h = jnp.dot(obs, w_gat_ref[...], preferred_element_type=jnp.float32)
    ones = jnp.ones((_N, 1), dtype=jnp.float32)
    h_ext = jnp.concatenate([h, ones], axis=1)

    s_col = jnp.dot(h, a_src_ref[...], preferred_element_type=jnp.float32)
    vd_row = jnp.dot(a_dst_ref[...], w_gat_t_ref[...],
                     preferred_element_type=jnp.float32)
    d_row = jnp.dot(vd_row, obs_t, preferred_element_type=jnp.float32)

    m = jnp.max(d_row)
    p_row = jnp.exp(d_row - m).astype(jnp.bfloat16)
    q_row = jnp.exp(0.2 * (d_row - m)).astype(jnp.bfloat16)

    sm = s_col + m
    big_m = jnp.maximum(sm, 0.2 * sm)
    u_col = jnp.exp(sm - big_m)
    v_col = jnp.exp(0.2 * sm - big_m)

    d_bf = (d_row - m).astype(jnp.bfloat16)
    neg_sm_bf = (-sm).astype(jnp.bfloat16)
    h_bf = h_ext.astype(jnp.bfloat16)

    zero_bf = jnp.zeros((), dtype=jnp.bfloat16)
    total = jnp.zeros((1, 24), dtype=jnp.float32)
    for ib in range(_N // _IB):
        sl = slice(ib * _IB, (ib + 1) * _IB)
        cond = d_bf >= neg_sm_bf[sl]
        wp = jnp.where(cond, jnp.broadcast_to(p_row, (_IB, _N)), zero_bf)
        wq = jnp.where(cond, zero_bf, jnp.broadcast_to(q_row, (_IB, _N)))
        accp = jnp.dot(wp, h_bf, preferred_element_type=jnp.float32)
        accq = jnp.dot(wq, h_bf, preferred_element_type=jnp.float32)
        acc = u_col[sl] * accp + v_col[sl] * accq
        o = acc[:, :24] / acc[:, 24:25]
        e = jnp.where(o > 0, o, jnp.exp(o) - 1.0)
        total = total + jnp.sum(e, axis=0, keepdims=True)

    mean = total * (1.0 / _N)
    z = jnp.maximum(jnp.dot(mean, w1_ref[...],
                            preferred_element_type=jnp.float32)
                    + b1_ref[...], 0.0)
    y = jnp.dot(z, w2_ref[...], preferred_element_type=jnp.float32) + b2_ref[...]
    out_ref[0] = jnp.broadcast_to(y, (8, 128))


def kernel(obs, W_gat, a_src, a_dst, W1, b1, W2, b2):
    B = obs.shape[0]
    obs_t = jnp.swapaxes(obs, 1, 2)
    a_src_col = a_src.reshape(24, 1)
    a_dst_row = a_dst.reshape(1, 24)
    b1_row = b1.reshape(1, 36)
    b2_s = b2.reshape(1, 1)

    grid_spec = pl.GridSpec(
        grid=(B,),
        in_specs=[
            pl.BlockSpec((1, _N, obs.shape[2]), lambda b: (b, 0, 0)),
            pl.BlockSpec((1, obs.shape[2], _N), lambda b: (b, 0, 0)),
            pl.BlockSpec(W_gat.shape, lambda b: (0, 0)),
            pl.BlockSpec(W_gat.T.shape, lambda b: (0, 0)),
            pl.BlockSpec((24, 1), lambda b: (0, 0)),
            pl.BlockSpec((1, 24), lambda b: (0, 0)),
            pl.BlockSpec(W1.shape, lambda b: (0, 0)),
            pl.BlockSpec((1, 36), lambda b: (0, 0)),
            pl.BlockSpec(W2.shape, lambda b: (0, 0)),
            pl.BlockSpec((1, 1), lambda b: (0, 0)),
        ],
        out_specs=pl.BlockSpec((1, 8, 128), lambda b: (b, 0, 0)),
    )
    padded = pl.pallas_call(
        _gat_value_kernel,
        grid_spec=grid_spec,
        out_shape=jax.ShapeDtypeStruct((B, 8, 128), jnp.float32),
        compiler_params=pltpu.CompilerParams(
            dimension_semantics=("arbitrary",),
        ),
    )(obs, obs_t, W_gat, W_gat.T, a_src_col, a_dst_row, W1, b1_row, W2, b2_s)
    return padded[:, 0, :1]


# speedup vs baseline: 2.8452x; 1.3303x over previous
import jax
import jax.numpy as jnp
from jax.experimental import pallas as pl
from jax.experimental.pallas import tpu as pltpu

_N = 2048
_IB = 256


def _gat_value_kernel(obs_ref, obs_t_ref, w_gat_ref, w_gat_t_ref,
                      a_src_ref, a_dst_ref, w1_ref, b1_ref, w2_ref, b2_ref,
                      out_ref):
    obs = obs_ref[0]
    obs_t = obs_t_ref[0]

    h = jnp.dot(obs, w_gat_ref[...], preferred_element_type=jnp.float32)
    ones = jnp.ones((_N, 1), dtype=jnp.float32)
    h_ext = jnp.concatenate([h, ones], axis=1)

    s_col = jnp.dot(h, a_src_ref[...], preferred_element_type=jnp.float32)
    vd_row = jnp.dot(a_dst_ref[...], w_gat_t_ref[...],
                     preferred_element_type=jnp.float32)
    d_row = jnp.dot(vd_row, obs_t, preferred_element_type=jnp.float32)

    m = jnp.max(d_row)
    p_row = jnp.exp(d_row - m).astype(jnp.bfloat16)
    q_row = jnp.exp(0.2 * (d_row - m)).astype(jnp.bfloat16)

    sm = s_col + m
    big_m = jnp.maximum(sm, 0.2 * sm)
    u_col = jnp.exp(sm - big_m).astype(jnp.bfloat16)
    v_col = jnp.exp(0.2 * sm - big_m).astype(jnp.bfloat16)

    d_bf = (d_row - m).astype(jnp.bfloat16)
    neg_sm_bf = (-sm).astype(jnp.bfloat16)
    h_bf = h_ext.astype(jnp.bfloat16)

    total = jnp.zeros((1, 24), dtype=jnp.float32)
    for ib in range(_N // _IB):
        sl = slice(ib * _IB, (ib + 1) * _IB)
        cond = d_bf >= neg_sm_bf[sl]
        w1 = jnp.where(cond, jnp.broadcast_to(p_row, (_IB, _N)),
                       jnp.broadcast_to(q_row, (_IB, _N)))
        w2 = jnp.where(cond, u_col[sl], v_col[sl])
        w = w1 * w2
        acc = jnp.dot(w, h_bf, preferred_element_type=jnp.float32)
        o = acc[:, :24] / acc[:, 24:25]
        e = jnp.where(o > 0, o, jnp.exp(o) - 1.0)
        total = total + jnp.sum(e, axis=0, keepdims=True)

    mean = total * (1.0 / _N)
    z = jnp.maximum(jnp.dot(mean, w1_ref[...],
                            preferred_element_type=jnp.float32)
                    + b1_ref[...], 0.0)
    y = jnp.dot(z, w2_ref[...], preferred_element_type=jnp.float32) + b2_ref[...]
    out_ref[0] = jnp.broadcast_to(y, (8, 128))


def kernel(obs, W_gat, a_src, a_dst, W1, b1, W2, b2):
    B = obs.shape[0]
    obs_t = jnp.swapaxes(obs, 1, 2)
    a_src_col = a_src.reshape(24, 1)
    a_dst_row = a_dst.reshape(1, 24)
    b1_row = b1.reshape(1, 36)
    b2_s = b2.reshape(1, 1)

    grid_spec = pl.GridSpec(
        grid=(B,),
        in_specs=[
            pl.BlockSpec((1, _N, obs.shape[2]), lambda b: (b, 0, 0)),
            pl.BlockSpec((1, obs.shape[2], _N), lambda b: (b, 0, 0)),
            pl.BlockSpec(W_gat.shape, lambda b: (0, 0)),
            pl.BlockSpec(W_gat.T.shape, lambda b: (0, 0)),
            pl.BlockSpec((24, 1), lambda b: (0, 0)),
            pl.BlockSpec((1, 24), lambda b: (0, 0)),
            pl.BlockSpec(W1.shape, lambda b: (0, 0)),
            pl.BlockSpec((1, 36), lambda b: (0, 0)),
            pl.BlockSpec(W2.shape, lambda b: (0, 0)),
            pl.BlockSpec((1, 1), lambda b: (0, 0)),
        ],
        out_specs=pl.BlockSpec((1, 8, 128), lambda b: (b, 0, 0)),
    )
    padded = pl.pallas_call(
        _gat_value_kernel,
        grid_spec=grid_spec,
        out_shape=jax.ShapeDtypeStruct((B, 8, 128), jnp.float32),
        compiler_params=pltpu.CompilerParams(
            dimension_semantics=("arbitrary",),
        ),
    )(obs, obs_t, W_gat, W_gat.T, a_src_col, a_dst_row, W1, b1_row, W2, b2_s)
    return padded[:, 0, :1]
